# baseline (speedup 1.0000x reference)
import functools

import jax
import jax.numpy as jnp
from jax import lax
from jax.experimental import pallas as pl
from jax.experimental.pallas import tpu as pltpu
from jax.experimental.pallas import tpu_sc as plsc

_LANES = 16
_NBUF = 8
_NOB = 2
_GROW = 16


def _mask_body(x_ref, mask_ref):
    mask_ref[...] = x_ref[...] == 0


def _make_mask(B, L, block_rows):
    return pl.pallas_call(
        _mask_body,
        grid=(B // block_rows,),
        in_specs=[pl.BlockSpec((block_rows, L), lambda i: (i, 0))],
        out_specs=pl.BlockSpec((block_rows, L), lambda i: (i, 0)),
        out_shape=jax.ShapeDtypeStruct((B, L), jnp.bool_),
    )


def _make_sc_combine(B, L, D):
    info = plsc.get_sparse_core_info()
    NC, NS = info.num_cores, info.num_subcores
    NW = NC * NS
    assert B % NW == 0
    rows_w = B // NW
    H0 = (L // 2) & ~7
    H1 = L - H0
    assert 0 < H0 <= 128 and 0 < H1 <= 128 and H1 % 8 == 0
    HMAX = max(H0, H1)
    assert rows_w % _GROW == 0
    NG = rows_w // _GROW
    NCHG = 2 * _GROW
    assert NCHG % _NBUF == 0
    NSUP = NCHG // _NBUF
    NSL = L // _LANES
    TAIL = L % _LANES
    mesh = plsc.VectorSubcoreMesh(core_axis_name="c", subcore_axis_name="s")

    @functools.partial(
        pl.kernel,
        mesh=mesh,
        compiler_params=pltpu.CompilerParams(
            use_tc_tiling_on_sc=False, needs_layout_passes=False),
        out_type=jax.ShapeDtypeStruct((B, L, D), jnp.float32),
        scratch_types=[
            pltpu.VMEM((_GROW, L), jnp.int32),
            pltpu.VMEM((_GROW, L), jnp.int32),
            pltpu.VMEM((_NBUF, HMAX, D), jnp.float32),
            pltpu.VMEM((_NBUF, HMAX, D), jnp.float32),
            pltpu.VMEM((_NOB, HMAX, D), jnp.float32),
            [pltpu.SemaphoreType.DMA] * _NBUF,
            [pltpu.SemaphoreType.DMA] * _NOB,
        ],
    )
    def sc_combine(xi_hbm, tok_hbm, pos_hbm, out_hbm,
                   xi_v, pi_v, tr, pr, ob, sg, so):
        wid = lax.axis_index("s") * NC + lax.axis_index("c")
        row_base = wid * rows_w

        lanes = lax.iota(jnp.int32, _LANES)

        def pos_row(r, carry):
            c = jnp.int32(0)
            for s in range(NSL):
                xv = xi_v[r, pl.ds(s * _LANES, _LANES)]
                m = xv != 0
                mi = jnp.where(m, 1, 0)
                cs = plsc.cumsum(mi) + c
                pi_v[r, pl.ds(s * _LANES, _LANES)] = jnp.where(m, cs, 0)
                c = c + jnp.sum(mi)
            if TAIL:
                off = L - _LANES
                xv = xi_v[r, pl.ds(off, _LANES)]
                m = xv != 0
                mi = jnp.where(m, 1, 0)
                head = jnp.sum(jnp.where(lanes < NSL * _LANES - off, mi, 0))
                cs = plsc.cumsum(mi) + (c - head)
                pi_v[r, pl.ds(off, _LANES)] = jnp.where(m, cs, 0)
            return carry

        def issue(j, b):
            r = j // 2
            off, h = (0, H0) if b % 2 == 0 else (H0, H1)
            pltpu.async_copy(
                tok_hbm.at[xi_v.at[r, pl.ds(off, h)]],
                tr.at[b, pl.ds(0, h)], sg[b])
            pltpu.async_copy(
                pos_hbm.at[pi_v.at[r, pl.ds(off, h)]],
                pr.at[b, pl.ds(0, h)], sg[b])

        def group_body(g, carry):
            grow = row_base + g * _GROW
            pltpu.sync_copy(xi_hbm.at[pl.ds(grow, _GROW)], xi_v)
            lax.fori_loop(0, _GROW, pos_row, 0)
            for b in range(_NBUF):
                issue(b, b)

            def super_body(jj, carry2):
                for b in range(_NBUF):
                    j = jj * _NBUF + b
                    jg = g * NCHG + j
                    b2 = b % _NOB
                    off, h = (0, H0) if b % 2 == 0 else (H0, H1)
                    pltpu.make_async_copy(
                        tok_hbm.at[xi_v.at[0, pl.ds(0, h)]],
                        tr.at[b, pl.ds(0, h)], sg[b]).wait()
                    pltpu.make_async_copy(
                        pos_hbm.at[pi_v.at[0, pl.ds(0, h)]],
                        pr.at[b, pl.ds(0, h)], sg[b]).wait()

                    @pl.when(jg >= _NOB)
                    def _():
                        pltpu.make_async_copy(
                            ob.at[b2, pl.ds(0, h)],
                            out_hbm.at[0, pl.ds(off, h)], so[b2]).wait()

                    @plsc.parallel_loop(0, h, unroll=4)
                    def _(r):
                        for cc in range(D // _LANES):
                            sl = pl.ds(cc * _LANES, _LANES)
                            ob[b2, r, sl] = tr[b, r, sl] + pr[b, r, sl]

                    pltpu.async_copy(
                        ob.at[b2, pl.ds(0, h)],
                        out_hbm.at[grow + j // 2, pl.ds(off, h)],
                        so[b2])

                    @pl.when(j + _NBUF < NCHG)
                    def _():
                        issue(j + _NBUF, b)
                return carry2

            lax.fori_loop(0, NSUP, super_body, 0)
            return carry

        lax.fori_loop(0, NG, group_body, 0)
        for b2 in range(_NOB):
            off, h = (0, H0) if b2 % 2 == 0 else (H0, H1)
            pltpu.make_async_copy(
                ob.at[b2, pl.ds(0, h)],
                out_hbm.at[0, pl.ds(off, h)], so[b2]).wait()

    return sc_combine


def kernel(x, tok_table, pos_table):
    B, L = x.shape
    V, D = tok_table.shape
    x32 = x.astype(jnp.int32)
    mask = _make_mask(B, L, 512)(x32)
    out = _make_sc_combine(B, L, D)(x32, tok_table, pos_table)
    return out, mask


# speedup vs baseline: 1.0239x; 1.0239x over previous
import functools

import jax
import jax.numpy as jnp
from jax import lax
from jax.experimental import pallas as pl
from jax.experimental.pallas import tpu as pltpu
from jax.experimental.pallas import tpu_sc as plsc

_LANES = 16
_NBUF = 4
_NOB = 2


def _mask_body(x_ref, mask_ref):
    mask_ref[...] = x_ref[...] == 0


def _make_mask(B, L, block_rows):
    return pl.pallas_call(
        _mask_body,
        grid=(B // block_rows,),
        in_specs=[pl.BlockSpec((block_rows, L), lambda i: (i, 0))],
        out_specs=pl.BlockSpec((block_rows, L), lambda i: (i, 0)),
        out_shape=jax.ShapeDtypeStruct((B, L), jnp.bool_),
    )


def _make_sc_combine(B, L, D):
    info = plsc.get_sparse_core_info()
    NC, NS = info.num_cores, info.num_subcores
    NW = NC * NS
    assert B % NW == 0
    rows_w = B // NW
    H0 = (L // 2) & ~7
    H1 = L - H0
    assert 0 < H0 <= 128 and 0 < H1 <= 128 and H1 % 8 == 0
    HMAX = max(H0, H1)
    NCH = 2 * rows_w
    assert NCH % _NBUF == 0
    NSUP = NCH // _NBUF
    NSL = L // _LANES
    TAIL = L % _LANES
    mesh = plsc.VectorSubcoreMesh(core_axis_name="c", subcore_axis_name="s")

    @functools.partial(
        pl.kernel,
        mesh=mesh,
        compiler_params=pltpu.CompilerParams(
            use_tc_tiling_on_sc=False, needs_layout_passes=False),
        out_type=jax.ShapeDtypeStruct((B, L, D), jnp.float32),
        scratch_types=[
            pltpu.VMEM((rows_w, L), jnp.int32),
            pltpu.VMEM((rows_w, L), jnp.int32),
            pltpu.VMEM((_NBUF, HMAX, D), jnp.float32),
            pltpu.VMEM((_NBUF, HMAX, D), jnp.float32),
            pltpu.VMEM((_NOB, HMAX, D), jnp.float32),
            [pltpu.SemaphoreType.DMA] * _NBUF,
            [pltpu.SemaphoreType.DMA] * _NOB,
        ],
    )
    def sc_combine(xi_hbm, tok_hbm, pos_hbm, out_hbm,
                   xi_v, pi_v, tr, pr, ob, sg, so):
        wid = lax.axis_index("s") * NC + lax.axis_index("c")
        row_base = wid * rows_w
        pltpu.sync_copy(xi_hbm.at[pl.ds(row_base, rows_w)], xi_v)

        lanes = lax.iota(jnp.int32, _LANES)

        def pos_row(r, carry):
            c = jnp.int32(0)
            for s in range(NSL):
                xv = xi_v[r, pl.ds(s * _LANES, _LANES)]
                m = xv != 0
                mi = jnp.where(m, 1, 0)
                cs = plsc.cumsum(mi) + c
                pi_v[r, pl.ds(s * _LANES, _LANES)] = jnp.where(m, cs, 0)
                c = c + jnp.sum(mi)
            if TAIL:
                off = L - _LANES
                xv = xi_v[r, pl.ds(off, _LANES)]
                m = xv != 0
                mi = jnp.where(m, 1, 0)
                head = jnp.sum(jnp.where(lanes < NSL * _LANES - off, mi, 0))
                cs = plsc.cumsum(mi) + (c - head)
                pi_v[r, pl.ds(off, _LANES)] = jnp.where(m, cs, 0)
            return carry

        lax.fori_loop(0, rows_w, pos_row, 0)

        def issue(j, b):
            r = j // 2
            off, h = (0, H0) if b % 2 == 0 else (H0, H1)
            pltpu.async_copy(
                tok_hbm.at[xi_v.at[r, pl.ds(off, h)]],
                tr.at[b, pl.ds(0, h)], sg[b])
            pltpu.async_copy(
                pos_hbm.at[pi_v.at[r, pl.ds(off, h)]],
                pr.at[b, pl.ds(0, h)], sg[b])

        for b in range(_NBUF):
            issue(b, b)

        def super_body(jj, carry2):
            for b in range(_NBUF):
                j = jj * _NBUF + b
                b2 = b % _NOB
                off, h = (0, H0) if b % 2 == 0 else (H0, H1)
                pltpu.make_async_copy(
                    tok_hbm.at[xi_v.at[0, pl.ds(0, h)]],
                    tr.at[b, pl.ds(0, h)], sg[b]).wait()
                pltpu.make_async_copy(
                    pos_hbm.at[pi_v.at[0, pl.ds(0, h)]],
                    pr.at[b, pl.ds(0, h)], sg[b]).wait()

                @pl.when(j >= _NOB)
                def _():
                    pltpu.make_async_copy(
                        ob.at[b2, pl.ds(0, h)],
                        out_hbm.at[0, pl.ds(off, h)], so[b2]).wait()

                @plsc.parallel_loop(0, h, unroll=4)
                def _(r):
                    for cc in range(D // _LANES):
                        sl = pl.ds(cc * _LANES, _LANES)
                        ob[b2, r, sl] = tr[b, r, sl] + pr[b, r, sl]

                pltpu.async_copy(
                    ob.at[b2, pl.ds(0, h)],
                    out_hbm.at[row_base + j // 2, pl.ds(off, h)],
                    so[b2])

                @pl.when(j + _NBUF < NCH)
                def _():
                    issue(j + _NBUF, b)
            return carry2

        lax.fori_loop(0, NSUP, super_body, 0)
        for b2 in range(_NOB):
            off, h = (0, H0) if b2 % 2 == 0 else (H0, H1)
            pltpu.make_async_copy(
                ob.at[b2, pl.ds(0, h)],
                out_hbm.at[0, pl.ds(off, h)], so[b2]).wait()

    return sc_combine


def kernel(x, tok_table, pos_table):
    B, L = x.shape
    V, D = tok_table.shape
    x32 = x.astype(jnp.int32)
    mask = _make_mask(B, L, 512)(x32)
    out = _make_sc_combine(B, L, D)(x32, tok_table, pos_table)
    return out, mask
